# baseline (speedup 1.0000x reference)
import functools

import jax
import jax.numpy as jnp
from jax import lax
from jax.experimental import pallas as pl
from jax.experimental.pallas import tpu as pltpu
from jax.experimental.pallas import tpu_sc as plsc

_N = 10000
_NT = 10112
_RPT = _NT // 16
_TRASH = _NT - 1
_B = 128
_NC = 2
_NS = 16
_F = 64
_H = 4


def _logit_table(alsd):
    mx4 = jnp.max(alsd[:, 0:4], axis=0, keepdims=True)
    t = mx4 + alsd[:, 4:8]
    c = jnp.maximum(t, 0.2 * t)
    return jnp.concatenate([alsd, c, jnp.zeros_like(c)], axis=1)


def _tc1_body(x_ref, w_ref, asd_ref, feat_ref, tab_ref):
    feat = jnp.dot(x_ref[...], w_ref[...], preferred_element_type=jnp.float32)
    feat_ref[...] = feat
    alsd = jnp.dot(feat, asd_ref[...], preferred_element_type=jnp.float32,
                   precision=lax.Precision.HIGHEST)
    tab_ref[...] = _logit_table(alsd)


def _combine_bn(acc_ref, b_ref, g_ref, be_ref, exp_ref):
    acc = acc_ref[0, :, 0:_F] + acc_ref[1, :, 0:_F]
    den = acc_ref[0, :, _F:_F + 4] + acc_ref[1, :, _F:_F + 4]
    dex = jnp.dot(den, exp_ref[...], preferred_element_type=jnp.float32,
                  precision=lax.Precision.HIGHEST)
    h = acc / (dex + 1e-16) + b_ref[...]
    mask = lax.broadcasted_iota(jnp.int32, (_NT, _F), 0) < _N
    hm = jnp.where(mask, h, 0.0)
    mean = jnp.sum(hm, axis=0, keepdims=True) * (1.0 / _N)
    cen = h - mean
    var = jnp.sum(jnp.where(mask, cen * cen, 0.0), axis=0, keepdims=True) * (
        1.0 / _N)
    hn = cen / jnp.sqrt(var + 1e-5) * g_ref[...] + be_ref[...]
    return jnp.where(mask, hn, 0.0)


def _tc2_body(acc_ref, b_ref, g_ref, be_ref, exp_ref, w_ref,
              asd_ref, feat_ref, tab_ref):
    hn = _combine_bn(acc_ref, b_ref, g_ref, be_ref, exp_ref)
    feat = jnp.dot(hn, w_ref[...], preferred_element_type=jnp.float32)
    feat_ref[...] = feat
    alsd = jnp.dot(feat, asd_ref[...], preferred_element_type=jnp.float32,
                   precision=lax.Precision.HIGHEST)
    tab_ref[...] = _logit_table(alsd)


def _tc3_body(acc_ref, b_ref, g_ref, be_ref, exp_ref, unp_ref, out_ref):
    acc = acc_ref[0, :, 0:_F] + acc_ref[1, :, 0:_F]
    den = acc_ref[0, :, _F:_F + 4] + acc_ref[1, :, _F:_F + 4]
    dex = jnp.dot(den, exp_ref[...], preferred_element_type=jnp.float32,
                  precision=lax.Precision.HIGHEST)
    hp = acc / (dex + 1e-16)
    h = jnp.dot(hp, unp_ref[...], preferred_element_type=jnp.float32,
                precision=lax.Precision.HIGHEST) + b_ref[...]
    mask = lax.broadcasted_iota(jnp.int32, (_NT, _F), 0) < _N
    hm = jnp.where(mask, h, 0.0)
    mean = jnp.sum(hm, axis=0, keepdims=True) * (1.0 / _N)
    cen = h - mean
    var = jnp.sum(jnp.where(mask, cen * cen, 0.0), axis=0, keepdims=True) * (
        1.0 / _N)
    out_ref[...] = cen / jnp.sqrt(var + 1e-5) * g_ref[...] + be_ref[...]


_W = _F + 16


_KI = 8


def _edge_body(n_chunks, feat, tab, srcp2, dstp2, z80,
               accp,
               acc_sh, *scr):
    c = lax.axis_index("c")
    s = lax.axis_index("s")
    wid = s * _NC + c
    sidx = scr[0:_KI]
    didx = scr[_KI:2 * _KI]
    hs = scr[2 * _KI:2 * _KI + 2]
    asg = scr[2 * _KI + 2:2 * _KI + 4]
    adg = scr[2 * _KI + 4:2 * _KI + 6]
    msg = scr[2 * _KI + 6:2 * _KI + 8]
    isem = scr[2 * _KI + 8:3 * _KI + 8]
    gsem = scr[3 * _KI + 8:3 * _KI + 10]
    ssem = scr[3 * _KI + 10:3 * _KI + 12]

    pltpu.sync_copy(z80, acc_sh.at[pl.ds(s * _RPT, _RPT)])
    plsc.subcore_barrier()

    iota = lax.iota(jnp.int32, 16)
    q = iota // _H
    r = iota - q * _H
    base = wid * n_chunks

    def i_descs(i, k):
        return (pltpu.make_async_copy(srcp2.at[base + i], sidx[k], isem[k]),
                pltpu.make_async_copy(dstp2.at[base + i], didx[k], isem[k]))

    def g_descs(b, k):
        return (pltpu.make_async_copy(feat.at[sidx[k]], hs[b], gsem[b]),
                pltpu.make_async_copy(tab.at[sidx[k]], asg[b], gsem[b]),
                pltpu.make_async_copy(tab.at[didx[k]], adg[b], gsem[b]))

    def s_desc(b, k):
        return pltpu.make_async_copy(msg[b], acc_sh.at[didx[k]], ssem[b])

    for k in range(_KI - 2):
        for dsc in i_descs(k, k):
            dsc.start()
    for b in range(2):
        for dsc in i_descs(b, b):
            dsc.wait()
        for dsc in g_descs(b, b):
            dsc.start()

    @pl.loop(0, n_chunks, step=_KI)
    def _pipe(i):
        for j in range(_KI):
            b = j % 2
            k = j
            ii = i + j
            for dsc in g_descs(b, k):
                dsc.wait()

            @pl.when(ii >= 2)
            def _():
                s_desc(b, (j - 2) % _KI).wait()

            @pl.when(ii + _KI - 2 < n_chunks)
            def _():
                for dsc in i_descs(ii + _KI - 2, (j - 2) % _KI):
                    dsc.start()

            def wgroup(g, carry2):
                row = g * 4 + q
                a_s = plsc.load_gather(asg[b], [row, r])
                a_d = plsc.load_gather(adg[b], [row, r + 4])
                cc = plsc.load_gather(adg[b], [row, r + 8])
                e = a_s + a_d
                e = jnp.maximum(e, 0.2 * e)
                w = jnp.exp(e - cc)
                for t in range(4):
                    plsc.store_scatter(msg[b], [row, r + _F + 4 * t], w)
                return carry2

            lax.fori_loop(0, _B // 4, wgroup, 0, unroll=2)

            def mrow(bb_, carry2):
                wv = msg[b][bb_, pl.ds(_F, 16)]
                for t in range(4):
                    msg[b][bb_, pl.ds(t * 16, 16)] = (
                        hs[b][bb_, pl.ds(t * 16, 16)] * wv)
                return carry2

            lax.fori_loop(0, _B, mrow, 0, unroll=4)

            s_desc(b, k).start(add=True)

            @pl.when(ii + 2 < n_chunks)
            def _():
                for dsc in i_descs(ii + 2, (j + 2) % _KI):
                    dsc.wait()
                for dsc in g_descs(b, (j + 2) % _KI):
                    dsc.start()

    for b in range(2):
        s_desc(b, (_KI - 2 + b) % _KI).wait()
    plsc.subcore_barrier()

    sl = pl.ds(s * _RPT, _RPT)
    pltpu.sync_copy(acc_sh.at[sl], accp.at[c, sl])


def _make_edge_kernel(n_chunks):
    mesh = plsc.VectorSubcoreMesh(
        core_axis_name="c", subcore_axis_name="s",
        num_cores=_NC, num_subcores=_NS)
    dma = pltpu.SemaphoreType.DMA
    i32, f32 = jnp.int32, jnp.float32
    return pl.kernel(
        functools.partial(_edge_body, n_chunks),
        out_type=jax.ShapeDtypeStruct((_NC, _NT, _W), f32),
        mesh=mesh,
        compiler_params=pltpu.CompilerParams(
            use_tc_tiling_on_sc=False, needs_layout_passes=False),
        scratch_types=(
            [pltpu.VMEM_SHARED((_NT, _W), f32)]
            + [pltpu.VMEM((_B,), i32) for _ in range(_KI)]
            + [pltpu.VMEM((_B,), i32) for _ in range(_KI)]
            + [pltpu.VMEM((_B, _F), f32) for _ in range(2)]
            + [pltpu.VMEM((_B, 16), f32) for _ in range(2)]
            + [pltpu.VMEM((_B, 16), f32) for _ in range(2)]
            + [pltpu.VMEM((_B, _W), f32) for _ in range(2)]
            + [dma for _ in range(_KI + 4)]
        ),
    )


def kernel(x, edge_index, W1, a1_src, a1_dst, b1, W2, a2_src, a2_dst, b2,
           bn1_gamma, bn1_beta, bn2_gamma, bn2_beta):
    e0 = edge_index.shape[1]
    e_tot = e0 + _N
    n_chunks = -(-e_tot // (_NC * _NS * _B))
    n_chunks = -(-n_chunks // _KI) * _KI
    e_pad = _NC * _NS * _B * n_chunks

    x_pad = jnp.zeros((_NT, 128), jnp.float32).at[:_N].set(x)
    loops = jnp.arange(_N, dtype=jnp.int32)
    padv = jnp.full((e_pad - e_tot,), _TRASH, jnp.int32)
    srcp = jnp.concatenate([edge_index[0], loops, padv]).reshape(-1, _B)
    dstp = jnp.concatenate([edge_index[1], loops, padv]).reshape(-1, _B)

    eye4 = jnp.eye(4, dtype=jnp.float32)
    As1 = (eye4[:, None, :] * a1_src[:, :, None]).reshape(_F, _H)
    Ad1 = (eye4[:, None, :] * a1_dst[:, :, None]).reshape(_F, _H)
    Asd1 = jnp.concatenate([As1, Ad1], axis=1)
    As2 = jnp.tile(a2_src.reshape(_F, 1), (1, _H))
    Ad2 = jnp.tile(a2_dst.reshape(_F, 1), (1, _H))
    Asd2 = jnp.concatenate([As2, Ad2], axis=1)
    pidx = jnp.array([(j % 4) * 16 + j // 4 for j in range(_F)], jnp.int32)
    W1p = W1[:, pidx]
    Asd1p = Asd1[pidx, :]
    W2p = W2[pidx, :][:, pidx]
    Asd2p = Asd2[pidx, :]
    ExpandP = jnp.tile(eye4, (1, 16))
    Unp = jnp.eye(_F, dtype=jnp.float32)[pidx]
    z80 = jnp.zeros((_RPT, _W), jnp.float32)
    r2 = lambda v: v.reshape(1, _F)
    r2p = lambda v: v.reshape(1, _F)[:, pidx]

    f32 = jnp.float32
    tc1 = pl.pallas_call(_tc1_body, out_shape=[
        jax.ShapeDtypeStruct((_NT, _F), f32),
        jax.ShapeDtypeStruct((_NT, 16), f32),
    ])
    tc2 = pl.pallas_call(_tc2_body, out_shape=[
        jax.ShapeDtypeStruct((_NT, _F), f32),
        jax.ShapeDtypeStruct((_NT, 16), f32),
    ])
    tc3 = pl.pallas_call(_tc3_body, out_shape=[
        jax.ShapeDtypeStruct((_NT, _F), f32),
    ])
    edge = _make_edge_kernel(n_chunks)

    feat1, tab1 = tc1(x_pad, W1p, Asd1p)
    acc1 = edge(feat1, tab1, srcp, dstp, z80)
    feat2, tab2 = tc2(acc1, r2p(b1), r2p(bn1_gamma),
                      r2p(bn1_beta), ExpandP, W2p, Asd2p)
    acc2 = edge(feat2, tab2, srcp, dstp, z80)
    (out,) = tc3(acc2, r2(b2), r2(bn2_gamma), r2(bn2_beta), ExpandP, Unp)
    return out[:_N]


# speedup vs baseline: 1.0064x; 1.0064x over previous
import functools

import jax
import jax.numpy as jnp
from jax import lax
from jax.experimental import pallas as pl
from jax.experimental.pallas import tpu as pltpu
from jax.experimental.pallas import tpu_sc as plsc

_N = 10000
_NT = 10112
_RPT = _NT // 16
_TRASH = _NT - 1
_B = 128
_NC = 2
_NS = 16
_F = 64
_H = 4


def _logit_table(alsd):
    mx4 = jnp.max(alsd[:, 0:4], axis=0, keepdims=True)
    t = mx4 + alsd[:, 4:8]
    c = jnp.maximum(t, 0.2 * t)
    return jnp.concatenate([alsd, c, jnp.zeros_like(c)], axis=1)


def _tc1_body(x_ref, w_ref, asd_ref, feat_ref, tab_ref):
    feat = jnp.dot(x_ref[...], w_ref[...], preferred_element_type=jnp.float32)
    feat_ref[...] = feat
    alsd = jnp.dot(feat, asd_ref[...], preferred_element_type=jnp.float32,
                   precision=lax.Precision.HIGHEST)
    tab_ref[...] = _logit_table(alsd)


def _combine_bn(acc_ref, b_ref, g_ref, be_ref, exp_ref):
    acc = acc_ref[0, :, 0:_F] + acc_ref[1, :, 0:_F]
    den = acc_ref[0, :, _F:_F + 4] + acc_ref[1, :, _F:_F + 4]
    dex = jnp.dot(den, exp_ref[...], preferred_element_type=jnp.float32,
                  precision=lax.Precision.HIGHEST)
    h = acc / (dex + 1e-16) + b_ref[...]
    mask = lax.broadcasted_iota(jnp.int32, (_NT, _F), 0) < _N
    hm = jnp.where(mask, h, 0.0)
    mean = jnp.sum(hm, axis=0, keepdims=True) * (1.0 / _N)
    cen = h - mean
    var = jnp.sum(jnp.where(mask, cen * cen, 0.0), axis=0, keepdims=True) * (
        1.0 / _N)
    hn = cen / jnp.sqrt(var + 1e-5) * g_ref[...] + be_ref[...]
    return jnp.where(mask, hn, 0.0)


def _tc2_body(acc_ref, b_ref, g_ref, be_ref, exp_ref, w_ref,
              asd_ref, feat_ref, tab_ref):
    hn = _combine_bn(acc_ref, b_ref, g_ref, be_ref, exp_ref)
    feat = jnp.dot(hn, w_ref[...], preferred_element_type=jnp.float32)
    feat_ref[...] = feat
    alsd = jnp.dot(feat, asd_ref[...], preferred_element_type=jnp.float32,
                   precision=lax.Precision.HIGHEST)
    tab_ref[...] = _logit_table(alsd)


def _tc3_body(acc_ref, b_ref, g_ref, be_ref, exp_ref, unp_ref, out_ref):
    acc = acc_ref[0, :, 0:_F] + acc_ref[1, :, 0:_F]
    den = acc_ref[0, :, _F:_F + 4] + acc_ref[1, :, _F:_F + 4]
    dex = jnp.dot(den, exp_ref[...], preferred_element_type=jnp.float32,
                  precision=lax.Precision.HIGHEST)
    hp = acc / (dex + 1e-16)
    h = jnp.dot(hp, unp_ref[...], preferred_element_type=jnp.float32,
                precision=lax.Precision.HIGHEST) + b_ref[...]
    mask = lax.broadcasted_iota(jnp.int32, (_NT, _F), 0) < _N
    hm = jnp.where(mask, h, 0.0)
    mean = jnp.sum(hm, axis=0, keepdims=True) * (1.0 / _N)
    cen = h - mean
    var = jnp.sum(jnp.where(mask, cen * cen, 0.0), axis=0, keepdims=True) * (
        1.0 / _N)
    out_ref[...] = cen / jnp.sqrt(var + 1e-5) * g_ref[...] + be_ref[...]


_W = _F + 16


_KI = 8


def _edge_body(n_chunks, feat, tab, srcp2, dstp2, z80,
               accp,
               acc_sh, *scr):
    c = lax.axis_index("c")
    s = lax.axis_index("s")
    wid = s * _NC + c
    sidx = scr[0:_KI]
    didx = scr[_KI:2 * _KI]
    hs = scr[2 * _KI:2 * _KI + 2]
    asg = scr[2 * _KI + 2:2 * _KI + 4]
    adg = scr[2 * _KI + 4:2 * _KI + 6]
    msg = scr[2 * _KI + 6:2 * _KI + 8]
    isem = scr[2 * _KI + 8:3 * _KI + 8]
    gsem = scr[3 * _KI + 8:3 * _KI + 10]
    ssem = scr[3 * _KI + 10:3 * _KI + 12]

    pltpu.sync_copy(z80, acc_sh.at[pl.ds(s * _RPT, _RPT)])
    pltpu.sync_copy(z80.at[pl.ds(0, _B)], msg[0])
    pltpu.sync_copy(z80.at[pl.ds(0, _B)], msg[1])
    plsc.subcore_barrier()

    iota = lax.iota(jnp.int32, 16)
    q = iota // _H
    r = iota - q * _H
    base = wid * n_chunks

    def i_descs(i, k):
        return (pltpu.make_async_copy(srcp2.at[base + i], sidx[k], isem[k]),
                pltpu.make_async_copy(dstp2.at[base + i], didx[k], isem[k]))

    def g_descs(b, k):
        return (pltpu.make_async_copy(feat.at[sidx[k]], hs[b], gsem[b]),
                pltpu.make_async_copy(tab.at[sidx[k]], asg[b], gsem[b]),
                pltpu.make_async_copy(tab.at[didx[k]], adg[b], gsem[b]))

    def s_desc(b, k):
        return pltpu.make_async_copy(msg[b], acc_sh.at[didx[k]], ssem[b])

    for k in range(_KI - 2):
        for dsc in i_descs(k, k):
            dsc.start()
    for b in range(2):
        for dsc in i_descs(b, b):
            dsc.wait()
        for dsc in g_descs(b, b):
            dsc.start()

    @pl.loop(0, n_chunks, step=_KI)
    def _pipe(i):
        for j in range(_KI):
            b = j % 2
            k = j
            ii = i + j
            for dsc in g_descs(b, k):
                dsc.wait()

            @pl.when(ii >= 2)
            def _():
                s_desc(b, (j - 2) % _KI).wait()

            @pl.when(ii + _KI - 2 < n_chunks)
            def _():
                for dsc in i_descs(ii + _KI - 2, (j - 2) % _KI):
                    dsc.start()

            def wgroup(g, carry2):
                row = g * 4 + q
                a_s = plsc.load_gather(asg[b], [row, r])
                a_d = plsc.load_gather(adg[b], [row, r + 4])
                cc = plsc.load_gather(adg[b], [row, r + 8])
                e = a_s + a_d
                e = jnp.maximum(e, 0.2 * e)
                w = jnp.exp(e - cc)
                plsc.store_scatter(msg[b], [row, r + _F], w)
                return carry2

            lax.fori_loop(0, _B // 4, wgroup, 0, unroll=2)

            wrep = _F + r

            def mrow(bb_, carry2):
                bb = jnp.full((16,), bb_, jnp.int32)
                wv = plsc.load_gather(msg[b], [bb, wrep])
                for t in range(4):
                    msg[b][bb_, pl.ds(t * 16, 16)] = (
                        hs[b][bb_, pl.ds(t * 16, 16)] * wv)
                return carry2

            lax.fori_loop(0, _B, mrow, 0, unroll=4)

            s_desc(b, k).start(add=True)

            @pl.when(ii + 2 < n_chunks)
            def _():
                for dsc in i_descs(ii + 2, (j + 2) % _KI):
                    dsc.wait()
                for dsc in g_descs(b, (j + 2) % _KI):
                    dsc.start()

    for b in range(2):
        s_desc(b, (_KI - 2 + b) % _KI).wait()
    plsc.subcore_barrier()

    sl = pl.ds(s * _RPT, _RPT)
    pltpu.sync_copy(acc_sh.at[sl], accp.at[c, sl])


def _make_edge_kernel(n_chunks):
    mesh = plsc.VectorSubcoreMesh(
        core_axis_name="c", subcore_axis_name="s",
        num_cores=_NC, num_subcores=_NS)
    dma = pltpu.SemaphoreType.DMA
    i32, f32 = jnp.int32, jnp.float32
    return pl.kernel(
        functools.partial(_edge_body, n_chunks),
        out_type=jax.ShapeDtypeStruct((_NC, _NT, _W), f32),
        mesh=mesh,
        compiler_params=pltpu.CompilerParams(
            use_tc_tiling_on_sc=False, needs_layout_passes=False),
        scratch_types=(
            [pltpu.VMEM_SHARED((_NT, _W), f32)]
            + [pltpu.VMEM((_B,), i32) for _ in range(_KI)]
            + [pltpu.VMEM((_B,), i32) for _ in range(_KI)]
            + [pltpu.VMEM((_B, _F), f32) for _ in range(2)]
            + [pltpu.VMEM((_B, 16), f32) for _ in range(2)]
            + [pltpu.VMEM((_B, 16), f32) for _ in range(2)]
            + [pltpu.VMEM((_B, _W), f32) for _ in range(2)]
            + [dma for _ in range(_KI + 4)]
        ),
    )


def kernel(x, edge_index, W1, a1_src, a1_dst, b1, W2, a2_src, a2_dst, b2,
           bn1_gamma, bn1_beta, bn2_gamma, bn2_beta):
    e0 = edge_index.shape[1]
    e_tot = e0 + _N
    n_chunks = -(-e_tot // (_NC * _NS * _B))
    n_chunks = -(-n_chunks // _KI) * _KI
    e_pad = _NC * _NS * _B * n_chunks

    x_pad = jnp.zeros((_NT, 128), jnp.float32).at[:_N].set(x)
    loops = jnp.arange(_N, dtype=jnp.int32)
    padv = jnp.full((e_pad - e_tot,), _TRASH, jnp.int32)
    srcp = jnp.concatenate([edge_index[0], loops, padv]).reshape(-1, _B)
    dstp = jnp.concatenate([edge_index[1], loops, padv]).reshape(-1, _B)

    eye4 = jnp.eye(4, dtype=jnp.float32)
    As1 = (eye4[:, None, :] * a1_src[:, :, None]).reshape(_F, _H)
    Ad1 = (eye4[:, None, :] * a1_dst[:, :, None]).reshape(_F, _H)
    Asd1 = jnp.concatenate([As1, Ad1], axis=1)
    As2 = jnp.tile(a2_src.reshape(_F, 1), (1, _H))
    Ad2 = jnp.tile(a2_dst.reshape(_F, 1), (1, _H))
    Asd2 = jnp.concatenate([As2, Ad2], axis=1)
    pidx = jnp.array([(j % 4) * 16 + j // 4 for j in range(_F)], jnp.int32)
    W1p = W1[:, pidx]
    Asd1p = Asd1[pidx, :]
    W2p = W2[pidx, :][:, pidx]
    Asd2p = Asd2[pidx, :]
    ExpandP = jnp.tile(eye4, (1, 16))
    Unp = jnp.eye(_F, dtype=jnp.float32)[pidx]
    z80 = jnp.zeros((_RPT, _W), jnp.float32)
    r2 = lambda v: v.reshape(1, _F)
    r2p = lambda v: v.reshape(1, _F)[:, pidx]

    f32 = jnp.float32
    tc1 = pl.pallas_call(_tc1_body, out_shape=[
        jax.ShapeDtypeStruct((_NT, _F), f32),
        jax.ShapeDtypeStruct((_NT, 16), f32),
    ])
    tc2 = pl.pallas_call(_tc2_body, out_shape=[
        jax.ShapeDtypeStruct((_NT, _F), f32),
        jax.ShapeDtypeStruct((_NT, 16), f32),
    ])
    tc3 = pl.pallas_call(_tc3_body, out_shape=[
        jax.ShapeDtypeStruct((_NT, _F), f32),
    ])
    edge = _make_edge_kernel(n_chunks)

    feat1, tab1 = tc1(x_pad, W1p, Asd1p)
    acc1 = edge(feat1, tab1, srcp, dstp, z80)
    feat2, tab2 = tc2(acc1, r2p(b1), r2p(bn1_gamma),
                      r2p(bn1_beta), ExpandP, W2p, Asd2p)
    acc2 = edge(feat2, tab2, srcp, dstp, z80)
    (out,) = tc3(acc2, r2(b2), r2(bn2_gamma), r2(bn2_beta), ExpandP, Unp)
    return out[:_N]


# speedup vs baseline: 1.0417x; 1.0350x over previous
import functools

import jax
import jax.numpy as jnp
from jax import lax
from jax.experimental import pallas as pl
from jax.experimental.pallas import tpu as pltpu
from jax.experimental.pallas import tpu_sc as plsc

_N = 10000
_NT = 10112
_RPT = _NT // 16
_TRASH = _NT - 1
_B = 128
_NC = 2
_NS = 16
_F = 64
_H = 4


def _logit_table(alsd):
    mx4 = jnp.max(alsd[:, 0:4], axis=0, keepdims=True)
    t = mx4 + alsd[:, 4:8]
    u = jnp.maximum(t, 0.2 * t)
    es = alsd[:, 0:4] + alsd[:, 4:8]
    es = jnp.maximum(es, 0.2 * es)
    c = jnp.maximum(es, u - 60.0)
    return jnp.concatenate([alsd, c, jnp.zeros_like(c)], axis=1)


def _tc1_body(x_ref, w_ref, asd_ref, feat_ref, tab_ref):
    feat = jnp.dot(x_ref[...], w_ref[...], preferred_element_type=jnp.float32)
    feat_ref[...] = feat
    alsd = jnp.dot(feat, asd_ref[...], preferred_element_type=jnp.float32,
                   precision=lax.Precision.HIGHEST)
    tab_ref[...] = _logit_table(alsd)


def _combine_bn(acc_ref, b_ref, g_ref, be_ref, exp_ref):
    acc = acc_ref[0, :, 0:_F] + acc_ref[1, :, 0:_F]
    den = acc_ref[0, :, _F:_F + 4] + acc_ref[1, :, _F:_F + 4]
    dex = jnp.dot(den, exp_ref[...], preferred_element_type=jnp.float32,
                  precision=lax.Precision.HIGHEST)
    h = acc / (dex + 1e-16) + b_ref[...]
    mask = lax.broadcasted_iota(jnp.int32, (_NT, _F), 0) < _N
    hm = jnp.where(mask, h, 0.0)
    mean = jnp.sum(hm, axis=0, keepdims=True) * (1.0 / _N)
    cen = h - mean
    var = jnp.sum(jnp.where(mask, cen * cen, 0.0), axis=0, keepdims=True) * (
        1.0 / _N)
    hn = cen / jnp.sqrt(var + 1e-5) * g_ref[...] + be_ref[...]
    return jnp.where(mask, hn, 0.0)


def _tc2_body(acc_ref, b_ref, g_ref, be_ref, exp_ref, w_ref,
              asd_ref, feat_ref, tab_ref):
    hn = _combine_bn(acc_ref, b_ref, g_ref, be_ref, exp_ref)
    feat = jnp.dot(hn, w_ref[...], preferred_element_type=jnp.float32)
    feat_ref[...] = feat
    alsd = jnp.dot(feat, asd_ref[...], preferred_element_type=jnp.float32,
                   precision=lax.Precision.HIGHEST)
    tab_ref[...] = _logit_table(alsd)


def _tc3_body(acc_ref, b_ref, g_ref, be_ref, exp_ref, out_ref):
    out_ref[...] = _combine_bn(acc_ref, b_ref, g_ref, be_ref, exp_ref)


_W = _F + 16


_KI = 8


def _edge_body(n_chunks, feat, tab, srcp2, dstp2, z80,
               accp,
               acc_sh, *scr):
    c = lax.axis_index("c")
    s = lax.axis_index("s")
    wid = s * _NC + c
    sidx = scr[0:_KI]
    didx = scr[_KI:2 * _KI]
    hs = scr[2 * _KI:2 * _KI + 2]
    asg = scr[2 * _KI + 2:2 * _KI + 4]
    adg = scr[2 * _KI + 4:2 * _KI + 6]
    msg = scr[2 * _KI + 6:2 * _KI + 8]
    isem = scr[2 * _KI + 8:3 * _KI + 8]
    gsem = scr[3 * _KI + 8:3 * _KI + 10]
    ssem = scr[3 * _KI + 10:3 * _KI + 12]

    pltpu.sync_copy(z80, acc_sh.at[pl.ds(s * _RPT, _RPT)])
    pltpu.sync_copy(z80.at[pl.ds(0, _B)], msg[0])
    pltpu.sync_copy(z80.at[pl.ds(0, _B)], msg[1])
    plsc.subcore_barrier()

    iota = lax.iota(jnp.int32, 16)
    q = iota // _H
    r = iota - q * _H
    base = wid * n_chunks

    def i_descs(i, k):
        return (pltpu.make_async_copy(srcp2.at[base + i], sidx[k], isem[k]),
                pltpu.make_async_copy(dstp2.at[base + i], didx[k], isem[k]))

    def g_descs(b, k):
        return (pltpu.make_async_copy(feat.at[sidx[k]], hs[b], gsem[b]),
                pltpu.make_async_copy(tab.at[sidx[k]], asg[b], gsem[b]),
                pltpu.make_async_copy(tab.at[didx[k]], adg[b], gsem[b]))

    def s_desc(b, k):
        return pltpu.make_async_copy(msg[b], acc_sh.at[didx[k]], ssem[b])

    for k in range(_KI - 2):
        for dsc in i_descs(k, k):
            dsc.start()
    for b in range(2):
        for dsc in i_descs(b, b):
            dsc.wait()
        for dsc in g_descs(b, b):
            dsc.start()

    @pl.loop(0, n_chunks, step=_KI)
    def _pipe(i):
        for j in range(_KI):
            b = j % 2
            k = j
            ii = i + j
            for dsc in g_descs(b, k):
                dsc.wait()

            @pl.when(ii >= 2)
            def _():
                s_desc(b, (j - 2) % _KI).wait()

            @pl.when(ii + _KI - 2 < n_chunks)
            def _():
                for dsc in i_descs(ii + _KI - 2, (j - 2) % _KI):
                    dsc.start()

            def wgroup(g, carry2):
                row = g * 4 + q
                a_s = plsc.load_gather(asg[b], [row, r])
                a_d = plsc.load_gather(adg[b], [row, r + 4])
                cc = plsc.load_gather(adg[b], [row, r + 8])
                e = a_s + a_d
                e = jnp.maximum(e, 0.2 * e)
                w = jnp.exp(e - cc)
                plsc.store_scatter(msg[b], [row, r + _F], w)
                return carry2

            lax.fori_loop(0, _B // 4, wgroup, 0, unroll=2)

            def mrow(bb_, carry2):
                wv = msg[b][bb_, pl.ds(_F, 16)]
                for h in range(_H):
                    msg[b][bb_, pl.ds(h * 16, 16)] = (
                        hs[b][bb_, pl.ds(h * 16, 16)] * wv[h])
                return carry2

            lax.fori_loop(0, _B, mrow, 0, unroll=4)

            s_desc(b, k).start(add=True)

            @pl.when(ii + 2 < n_chunks)
            def _():
                for dsc in i_descs(ii + 2, (j + 2) % _KI):
                    dsc.wait()
                for dsc in g_descs(b, (j + 2) % _KI):
                    dsc.start()

    for b in range(2):
        s_desc(b, (_KI - 2 + b) % _KI).wait()
    plsc.subcore_barrier()

    sl = pl.ds(s * _RPT, _RPT)
    pltpu.sync_copy(acc_sh.at[sl], accp.at[c, sl])


def _make_edge_kernel(n_chunks):
    mesh = plsc.VectorSubcoreMesh(
        core_axis_name="c", subcore_axis_name="s",
        num_cores=_NC, num_subcores=_NS)
    dma = pltpu.SemaphoreType.DMA
    i32, f32 = jnp.int32, jnp.float32
    return pl.kernel(
        functools.partial(_edge_body, n_chunks),
        out_type=jax.ShapeDtypeStruct((_NC, _NT, _W), f32),
        mesh=mesh,
        compiler_params=pltpu.CompilerParams(
            use_tc_tiling_on_sc=False, needs_layout_passes=False),
        scratch_types=(
            [pltpu.VMEM_SHARED((_NT, _W), f32)]
            + [pltpu.VMEM((_B,), i32) for _ in range(_KI)]
            + [pltpu.VMEM((_B,), i32) for _ in range(_KI)]
            + [pltpu.VMEM((_B, _F), f32) for _ in range(2)]
            + [pltpu.VMEM((_B, 16), f32) for _ in range(2)]
            + [pltpu.VMEM((_B, 16), f32) for _ in range(2)]
            + [pltpu.VMEM((_B, _W), f32) for _ in range(2)]
            + [dma for _ in range(_KI + 4)]
        ),
    )


def kernel(x, edge_index, W1, a1_src, a1_dst, b1, W2, a2_src, a2_dst, b2,
           bn1_gamma, bn1_beta, bn2_gamma, bn2_beta):
    e0 = edge_index.shape[1]
    e_tot = e0 + _N
    n_chunks = -(-e_tot // (_NC * _NS * _B))
    n_chunks = -(-n_chunks // _KI) * _KI
    e_pad = _NC * _NS * _B * n_chunks

    x_pad = jnp.zeros((_NT, 128), jnp.float32).at[:_N].set(x)
    loops = jnp.arange(_N, dtype=jnp.int32)
    padv = jnp.full((e_pad - e_tot,), _TRASH, jnp.int32)
    srcp = jnp.concatenate([edge_index[0], loops, padv]).reshape(-1, _B)
    dstp = jnp.concatenate([edge_index[1], loops, padv]).reshape(-1, _B)

    eye4 = jnp.eye(4, dtype=jnp.float32)
    As1 = (eye4[:, None, :] * a1_src[:, :, None]).reshape(_F, _H)
    Ad1 = (eye4[:, None, :] * a1_dst[:, :, None]).reshape(_F, _H)
    Asd1 = jnp.concatenate([As1, Ad1], axis=1)
    As2 = jnp.tile(a2_src.reshape(_F, 1), (1, _H))
    Ad2 = jnp.tile(a2_dst.reshape(_F, 1), (1, _H))
    Asd2 = jnp.concatenate([As2, Ad2], axis=1)
    Expand = jnp.repeat(eye4, 16, axis=1)
    Expand2 = Expand * 0.0 + 0.25
    z80 = jnp.zeros((_RPT, _W), jnp.float32)
    r2 = lambda v: v.reshape(1, _F)

    f32 = jnp.float32
    tc1 = pl.pallas_call(_tc1_body, out_shape=[
        jax.ShapeDtypeStruct((_NT, _F), f32),
        jax.ShapeDtypeStruct((_NT, 16), f32),
    ])
    tc2 = pl.pallas_call(_tc2_body, out_shape=[
        jax.ShapeDtypeStruct((_NT, _F), f32),
        jax.ShapeDtypeStruct((_NT, 16), f32),
    ])
    tc3 = pl.pallas_call(_tc3_body, out_shape=[
        jax.ShapeDtypeStruct((_NT, _F), f32),
    ])
    edge = _make_edge_kernel(n_chunks)

    feat1, tab1 = tc1(x_pad, W1, Asd1)
    acc1 = edge(feat1, tab1, srcp, dstp, z80)
    feat2, tab2 = tc2(acc1, r2(b1), r2(bn1_gamma),
                      r2(bn1_beta), Expand, W2, Asd2)
    acc2 = edge(feat2, tab2, srcp, dstp, z80)
    (out,) = tc3(acc2, r2(b2), r2(bn2_gamma), r2(bn2_beta), Expand2)
    return out[:_N]
